# baseline (speedup 1.0000x reference)
import functools

import jax
import jax.numpy as jnp
from jax import lax
from jax.experimental import pallas as pl
from jax.experimental.pallas import tpu as pltpu
from jax.experimental.pallas import tpu_sc as plsc

NC = 2
NS = 16
NW = NC * NS

IDXW = 128
GPC = 8
CHUNK = IDXW * GPC


def _emb_body(ids_hbm, table_hbm, out_hbm, idx_v, rows_v, sem):
    d = table_hbm.shape[1]
    b_per_w = out_hbm.shape[0] // NW
    n_chunks = b_per_w // CHUNK
    wid = lax.axis_index("s") * NC + lax.axis_index("c")
    row0 = wid * (b_per_w // IDXW)

    def body(j, _):
        base = wid * b_per_w + j * CHUNK
        pltpu.sync_copy(ids_hbm.at[pl.ds(row0 + j * GPC, GPC)], idx_v)
        copies = [
            pltpu.async_copy(
                table_hbm.at[idx_v.at[r]],
                rows_v.at[pl.ds(r * IDXW, IDXW)],
                sem,
            )
            for r in range(GPC)
        ]
        for c in copies:
            c.wait()
        pltpu.sync_copy(rows_v, out_hbm.at[pl.ds(base, CHUNK)])
        return 0

    lax.fori_loop(0, n_chunks, body, 0)


@functools.partial(jax.jit, static_argnames=())
def kernel(input_ids, table):
    batch, seq_len = input_ids.shape
    d = table.shape[1]
    b = batch * seq_len
    ids2d = input_ids.reshape(b // IDXW, IDXW).astype(jnp.int32)

    mesh = plsc.VectorSubcoreMesh(core_axis_name="c", subcore_axis_name="s")
    out = pl.kernel(
        _emb_body,
        out_type=jax.ShapeDtypeStruct((b, d), jnp.float32),
        mesh=mesh,
        scratch_types=[
            pltpu.VMEM((GPC, IDXW), jnp.int32),
            pltpu.VMEM((CHUNK, d), jnp.float32),
            pltpu.SemaphoreType.DMA,
        ],
        compiler_params=pltpu.CompilerParams(use_tc_tiling_on_sc=False),
    )(ids2d, table)
    return out.reshape(batch, seq_len, d)


# speedup vs baseline: 1.0198x; 1.0198x over previous
import functools

import jax
import jax.numpy as jnp
from jax import lax
from jax.experimental import pallas as pl
from jax.experimental.pallas import tpu as pltpu
from jax.experimental.pallas import tpu_sc as plsc

NC = 2
NS = 16
NW = NC * NS

IDXW = 128
GPC = 4
CHUNK = IDXW * GPC
NBUF = 2


def _emb_body(ids_hbm, table_hbm, out_hbm, idx_v, rows_v, sem_g, sem_o):
    d = table_hbm.shape[1]
    b_total = out_hbm.shape[0]
    b_per_w = b_total // NW
    n_chunks = b_per_w // CHUNK
    n_grp = n_chunks // NBUF
    wid = lax.axis_index("s") * NC + lax.axis_index("c")
    row0 = wid * (b_per_w // IDXW)
    base0 = wid * b_per_w

    def start_gather(j, b):
        pltpu.sync_copy(ids_hbm.at[pl.ds(row0 + j * GPC, GPC)], idx_v.at[b])
        for r in range(GPC):
            pltpu.async_copy(
                table_hbm.at[idx_v.at[b].at[r]],
                rows_v.at[b].at[pl.ds(r * IDXW, IDXW)],
                sem_g[b],
            )

    def wait_gather(b):
        pltpu.make_async_copy(
            table_hbm.at[pl.ds(0, CHUNK)], rows_v.at[b], sem_g[b]
        ).wait()

    def start_wb(j, b):
        pltpu.async_copy(rows_v.at[b], out_hbm.at[pl.ds(base0 + j * CHUNK, CHUNK)], sem_o[b])

    def wait_wb(b):
        pltpu.make_async_copy(
            rows_v.at[b], out_hbm.at[pl.ds(0, CHUNK)], sem_o[b]
        ).wait()

    for b in range(NBUF):
        start_gather(b, b)

    def grp(g, _):
        for b in range(NBUF):
            j = g * NBUF + b
            wait_gather(b)
            start_wb(j, b)
            wait_wb(b)
            start_gather(j + NBUF, b)
        return 0

    lax.fori_loop(0, n_grp - 1, grp, 0)

    for b in range(NBUF):
        j = (n_grp - 1) * NBUF + b
        wait_gather(b)
        start_wb(j, b)
        wait_wb(b)


@functools.partial(jax.jit, static_argnames=())
def kernel(input_ids, table):
    batch, seq_len = input_ids.shape
    d = table.shape[1]
    b = batch * seq_len
    ids2d = input_ids.reshape(b // IDXW, IDXW).astype(jnp.int32)

    mesh = plsc.VectorSubcoreMesh(core_axis_name="c", subcore_axis_name="s")
    out = pl.kernel(
        _emb_body,
        out_type=jax.ShapeDtypeStruct((b, d), jnp.float32),
        mesh=mesh,
        scratch_types=[
            pltpu.VMEM((NBUF, GPC, IDXW), jnp.int32),
            pltpu.VMEM((NBUF, CHUNK, d), jnp.float32),
            [pltpu.SemaphoreType.DMA] * NBUF,
            [pltpu.SemaphoreType.DMA] * NBUF,
        ],
        compiler_params=pltpu.CompilerParams(use_tc_tiling_on_sc=False),
    )(ids2d, table)
    return out.reshape(batch, seq_len, d)
